# baseline (speedup 1.0000x reference)
import functools

import jax
import jax.numpy as jnp
from jax import lax
from jax.experimental import pallas as pl
from jax.experimental.pallas import tpu as pltpu
from jax.experimental.pallas import tpu_sc as plsc

SEQ_BLOCK = 512


def _add_kernel(x_ref, pos_ref, o_ref):
    pos = pos_ref[...]
    o_ref[...] = x_ref[...] + pos[:, None, :]


def _kernel_tc(x, pos_embed_weight):
    S, B, D = x.shape
    pos = pos_embed_weight[:S]
    return pl.pallas_call(
        _add_kernel,
        grid=(S // SEQ_BLOCK,),
        in_specs=[
            pl.BlockSpec((SEQ_BLOCK, B, D), lambda i: (i, 0, 0)),
            pl.BlockSpec((SEQ_BLOCK, D), lambda i: (i, 0)),
        ],
        out_specs=pl.BlockSpec((SEQ_BLOCK, B, D), lambda i: (i, 0, 0)),
        out_shape=jax.ShapeDtypeStruct((S, B, D), x.dtype),
    )(x, pos)


_L = 16
_CHUNK = 8


def _sc_body(x_hbm, pos_hbm, out_hbm, xv, pv):
    S, B, D = x_hbm.shape
    nc = 2
    ns = 16
    rows_per_w = S // (nc * ns)
    wid = lax.axis_index("s") * nc + lax.axis_index("c")
    base = wid * rows_per_w

    def chunk_body(i, carry):
        row0 = base + i * _CHUNK
        pltpu.sync_copy(x_hbm.at[pl.ds(row0, _CHUNK)], xv)
        pltpu.sync_copy(pos_hbm.at[pl.ds(row0, _CHUNK)], pv)

        def j_body(j, c2):
            off = j * _L
            for r in range(_CHUNK):
                p = pv[r, pl.ds(off, _L)]
                for b in range(B):
                    xv[r, b, pl.ds(off, _L)] = xv[r, b, pl.ds(off, _L)] + p
            return c2

        lax.fori_loop(0, D // _L, j_body, 0)
        pltpu.sync_copy(xv, out_hbm.at[pl.ds(row0, _CHUNK)])
        return carry

    lax.fori_loop(0, (S // (nc * ns)) // _CHUNK, chunk_body, 0)


def _kernel_sc(x, pos_embed_weight):
    S, B, D = x.shape
    pos = pos_embed_weight[:S]
    mesh = plsc.VectorSubcoreMesh(core_axis_name="c", subcore_axis_name="s")
    run = pl.kernel(
        _sc_body,
        out_type=jax.ShapeDtypeStruct((S, B, D), x.dtype),
        mesh=mesh,
        scratch_types=[
            pltpu.VMEM((_CHUNK, B, D), jnp.float32),
            pltpu.VMEM((_CHUNK, D), jnp.float32),
        ],
    )
    return run(x, pos)


kernel = _kernel_sc
